# baseline (speedup 1.0000x reference)
import functools

import jax
import jax.numpy as jnp
from jax import lax
from jax.experimental import pallas as pl
from jax.experimental.pallas import tpu as pltpu
from jax.experimental.pallas import tpu_sc as plsc

D_MODEL_K = 1024
SCALE = 32.0
BATCH = 4
SEQ = 4096
NC = 2
NS = 16
NW = NC * NS
SPW = SEQ // NW
CHUNK = 8
NT = SPW // CHUNK
LANES = 16
COLS = D_MODEL_K // LANES
XROWS = BATCH * SEQ // CHUNK


def _make_kernel():
    mesh = plsc.VectorSubcoreMesh(core_axis_name="c", subcore_axis_name="s")

    @functools.partial(
        pl.kernel,
        mesh=mesh,
        out_type=jax.ShapeDtypeStruct((BATCH * SEQ, D_MODEL_K), jnp.float32),
        scratch_types=[
            pltpu.VMEM((BATCH, NT, CHUNK), jnp.int32),
            pltpu.VMEM((12, CHUNK, D_MODEL_K), jnp.float32),
            pltpu.VMEM((2, CHUNK, D_MODEL_K), jnp.float32),
            pltpu.SemaphoreType.DMA((12,)),
            pltpu.SemaphoreType.DMA((12,)),
            pltpu.SemaphoreType.DMA((2,)),
        ],
    )
    def k(x_hbm, table_hbm, enc_hbm, out_hbm, idx_v, rows_v, enc_v,
          gsem, ssem, esem):
        cid = lax.axis_index("c")
        sid = lax.axis_index("s")
        wid = sid * NC + cid
        seq0 = wid * SPW

        for b in range(BATCH):
            pltpu.sync_copy(
                x_hbm.at[pl.ds(b * (SEQ // CHUNK) + wid * NT, NT)],
                idx_v.at[b])

        def rbuf(t, b):
            return 4 * (t % 3) + b

        def issue_gather(t, b):
            pltpu.async_copy(table_hbm.at[idx_v.at[b, t]],
                             rows_v.at[rbuf(t, b)], gsem.at[rbuf(t, b)])

        def wait_gather(t, b):
            pltpu.make_async_copy(
                table_hbm.at[pl.ds(0, CHUNK)], rows_v.at[rbuf(t, b)],
                gsem.at[rbuf(t, b)]).wait()

        def issue_enc(t, e):
            pltpu.async_copy(enc_hbm.at[pl.ds(seq0 + t * CHUNK, CHUNK)],
                             enc_v.at[e], esem.at[e])

        def wait_enc(e):
            pltpu.make_async_copy(
                enc_hbm.at[pl.ds(0, CHUNK)], enc_v.at[e], esem.at[e]).wait()

        def issue_store(t, b):
            off = b * SEQ + seq0 + t * CHUNK
            pltpu.async_copy(rows_v.at[rbuf(t, b)],
                             out_hbm.at[pl.ds(off, CHUNK)],
                             ssem.at[rbuf(t, b)])

        def wait_store(t, b):
            pltpu.make_async_copy(
                rows_v.at[rbuf(t, b)], out_hbm.at[pl.ds(0, CHUNK)],
                ssem.at[rbuf(t, b)]).wait()

        def compute_step(t, e):
            banks = [rbuf(t, b) for b in range(BATCH)]

            def row_body(i, _):
                def col_body(j, _):
                    sl = pl.ds(pl.multiple_of(j * LANES, LANES), LANES)
                    ev = enc_v[e, i, sl]
                    for r in banks:
                        rows_v[r, i, sl] = rows_v[r, i, sl] * SCALE + ev
                    return 0

                lax.fori_loop(0, COLS, col_body, 0, unroll=8)
                return 0

            lax.fori_loop(0, CHUNK, row_body, 0)

        issue_enc(0, 0)
        issue_enc(1, 1)
        for b in range(BATCH):
            issue_gather(0, b)
        for b in range(BATCH):
            issue_gather(1, b)

        for t in range(NT):
            e = t % 2
            wait_enc(e)
            if t + 2 < NT:
                for b in range(BATCH):
                    if t >= 1:
                        wait_store(t - 1, b)
                    issue_gather(t + 2, b)
            for b in range(BATCH):
                wait_gather(t, b)
            compute_step(t, e)
            for b in range(BATCH):
                issue_store(t, b)
            if t + 2 < NT:
                issue_enc(t + 2, e)

        for t in (NT - 3, NT - 2, NT - 1):
            for b in range(BATCH):
                wait_store(t, b)

    return k


_sc_embed = _make_kernel()


def kernel(x, table, encoding):
    x_idx = x.reshape(XROWS, CHUNK).astype(jnp.int32)
    out = _sc_embed(x_idx, table, encoding)
    return out.reshape(x.shape[0], x.shape[1], D_MODEL_K)


# speedup vs baseline: 1.0749x; 1.0749x over previous
import functools

import jax
import jax.numpy as jnp
from jax import lax
from jax.experimental import pallas as pl
from jax.experimental.pallas import tpu as pltpu
from jax.experimental.pallas import tpu_sc as plsc

D_MODEL_K = 1024
SCALE = 32.0
BATCH = 4
SEQ = 4096
NC = 2
NS = 16
NW = NC * NS
SPW = SEQ // NW
CHUNK = 8
NT = SPW // CHUNK
LANES = 16
COLS = D_MODEL_K // LANES
XROWS = BATCH * SEQ // CHUNK


def _make_kernel():
    mesh = plsc.VectorSubcoreMesh(core_axis_name="c", subcore_axis_name="s")

    @functools.partial(
        pl.kernel,
        mesh=mesh,
        out_type=jax.ShapeDtypeStruct((BATCH * SEQ, D_MODEL_K), jnp.float32),
        scratch_types=[
            pltpu.VMEM((BATCH * NT, CHUNK), jnp.int32),
            pltpu.VMEM((8, CHUNK, D_MODEL_K), jnp.float32),
            pltpu.VMEM((2, CHUNK, D_MODEL_K), jnp.float32),
            pltpu.SemaphoreType.DMA((8,)),
            pltpu.SemaphoreType.DMA((8,)),
            pltpu.SemaphoreType.DMA((2,)),
        ],
    )
    def k(x_hbm, table_hbm, enc_hbm, out_hbm, idx_v, rows_v, enc_v,
          gsem, ssem, esem):
        cid = lax.axis_index("c")
        sid = lax.axis_index("s")
        wid = sid * NC + cid
        seq0 = wid * SPW

        def issue_enc_pre(t, e):
            pltpu.async_copy(enc_hbm.at[pl.ds(seq0 + t * CHUNK, CHUNK)],
                             enc_v.at[e], esem.at[e])

        issue_enc_pre(0, 0)
        issue_enc_pre(1, 1)

        pltpu.sync_copy(x_hbm.at[pl.ds(wid * BATCH * NT, BATCH * NT)], idx_v)

        def rbuf(t_par, b):
            return 4 * t_par + b

        def issue_gather(t, t_par, b):
            pltpu.async_copy(table_hbm.at[idx_v.at[b * NT + t]],
                             rows_v.at[rbuf(t_par, b)], gsem.at[rbuf(t_par, b)])

        def wait_gather(t_par, b):
            pltpu.make_async_copy(
                table_hbm.at[pl.ds(0, CHUNK)], rows_v.at[rbuf(t_par, b)],
                gsem.at[rbuf(t_par, b)]).wait()

        def issue_enc(t, e):
            pltpu.async_copy(enc_hbm.at[pl.ds(seq0 + t * CHUNK, CHUNK)],
                             enc_v.at[e], esem.at[e])

        def wait_enc(e):
            pltpu.make_async_copy(
                enc_hbm.at[pl.ds(0, CHUNK)], enc_v.at[e], esem.at[e]).wait()

        def issue_store(t, t_par, b):
            off = b * SEQ + seq0 + t * CHUNK
            pltpu.async_copy(rows_v.at[rbuf(t_par, b)],
                             out_hbm.at[pl.ds(off, CHUNK)],
                             ssem.at[rbuf(t_par, b)])

        def wait_store(t_par, b):
            pltpu.make_async_copy(
                rows_v.at[rbuf(t_par, b)], out_hbm.at[pl.ds(0, CHUNK)],
                ssem.at[rbuf(t_par, b)]).wait()

        def compute_step(t_par, e):
            banks = [rbuf(t_par, b) for b in range(BATCH)]

            def row_body(i, _):
                def col_body(j, _):
                    sl = pl.ds(pl.multiple_of(j * LANES, LANES), LANES)
                    ev = enc_v[e, i, sl]
                    for r in banks:
                        rows_v[r, i, sl] = rows_v[r, i, sl] * SCALE + ev
                    return 0

                lax.fori_loop(0, COLS, col_body, 0, unroll=8)
                return 0

            lax.fori_loop(0, CHUNK, row_body, 0)

        for b in range(BATCH):
            issue_gather(0, 0, b)

        def step(tt, _):
            for par in (0, 1):
                t = tt * 2 + par
                e = par
                wait_enc(e)
                for b in range(BATCH):
                    if par == 0:
                        @pl.when(t > 0)
                        def _():
                            wait_store(1, b)
                    else:
                        wait_store(0, b)
                    @pl.when(t + 1 < NT)
                    def _():
                        issue_gather(t + 1, 1 - par, b)
                for b in range(BATCH):
                    wait_gather(par, b)
                compute_step(par, e)
                for b in range(BATCH):
                    issue_store(t, par, b)
                @pl.when(t + 2 < NT)
                def _():
                    issue_enc(t + 2, e)
            return 0

        lax.fori_loop(0, NT // 2, step, 0)
        for b in range(BATCH):
            wait_store(1, b)

    return k


_sc_embed = _make_kernel()


def kernel(x, table, encoding):
    x_idx = (x.reshape(BATCH, NW, NT * CHUNK)
             .transpose(1, 0, 2)
             .reshape(XROWS, CHUNK)
             .astype(jnp.int32))
    out = _sc_embed(x_idx, table, encoding)
    return out.reshape(x.shape[0], x.shape[1], D_MODEL_K)
